# baseline (speedup 1.0000x reference)
import functools

import jax
import jax.numpy as jnp
from jax import lax
from jax.experimental import pallas as pl
from jax.experimental.pallas import tpu as pltpu
from jax.experimental.pallas import tpu_sc as plsc

NC, NS, L = 2, 16, 16
NW = NC * NS

ROWS_PER_CHUNK = 4
RING = 3
PREF_D = 2
PARTS_PAD = 8


def _sc_gather_loss(table, idx2, tgt2, n_rows, vocab):
    n_chunks = n_rows // ROWS_PER_CHUNK
    cpw = n_chunks // NW
    steps = vocab // L

    mesh = plsc.VectorSubcoreMesh(
        core_axis_name="c", subcore_axis_name="s",
        num_cores=NC, num_subcores=NS)

    @functools.partial(
        pl.kernel,
        out_type=(
            jax.ShapeDtypeStruct((n_rows, vocab), jnp.float32),
            jax.ShapeDtypeStruct((n_chunks, PARTS_PAD, L), jnp.float32),
        ),
        mesh=mesh,
        compiler_params=pltpu.CompilerParams(needs_layout_passes=False),
        scratch_types=(
            [pltpu.VMEM((cpw, ROWS_PER_CHUNK), jnp.int32),
             pltpu.VMEM((cpw * ROWS_PER_CHUNK,), jnp.int32),
             pltpu.VMEM((RING, PARTS_PAD, L), jnp.float32)]
            + [pltpu.VMEM((ROWS_PER_CHUNK, vocab), jnp.float32)] * RING
            + [pltpu.SemaphoreType.DMA] * (2 * RING)
        ),
    )
    def body(table_hbm, idx_hbm, tgt_hbm, logits_hbm, parts_hbm, *scratch):
        idx_v, tgt_v, parts_v = scratch[:3]
        bufs = scratch[3:3 + RING]
        gsems = scratch[3 + RING:3 + 2 * RING]
        ssems = scratch[3 + 2 * RING:3 + 3 * RING]

        w = lax.axis_index("s") * NC + lax.axis_index("c")
        cbase = w * cpw

        pltpu.sync_copy(idx_hbm.at[pl.ds(cbase, cpw)], idx_v)
        pltpu.sync_copy(
            tgt_hbm.at[pl.ds(cbase * ROWS_PER_CHUNK, cpw * ROWS_PER_CHUNK)],
            tgt_v)

        lane = lax.iota(jnp.int32, L)
        maskr = lane < ROWS_PER_CHUNK

        def start_gather(c, k):
            pltpu.async_copy(table_hbm.at[idx_v.at[c]], bufs[k], gsems[k])

        def wait_gather(c, k):
            pltpu.make_async_copy(
                table_hbm.at[idx_v.at[c]], bufs[k], gsems[k]).wait()

        def logits_dst(c):
            return logits_hbm.at[pl.ds((cbase + c) * ROWS_PER_CHUNK,
                                       ROWS_PER_CHUNK)]

        def wait_scatter(c, k):
            pltpu.make_async_copy(bufs[k], logits_dst(c), ssems[k]).wait()

        for m in range(PREF_D):
            start_gather(m, m)

        def do_chunk(c, k, prefetch, slot):
            buf = bufs[k]
            wait_gather(c, k)

            pltpu.async_copy(buf, logits_dst(c), ssems[k])
            if prefetch:
                k2 = (k + PREF_D) % RING
                @pl.when(c >= RING - PREF_D)
                def _():
                    wait_scatter(c + PREF_D - RING, k2)
                start_gather(c + PREF_D, k2)

            def inner(i, accs):
                s = pl.ds(i * L, L)
                return tuple(a + jnp.exp(buf[j, s]) for j, a in enumerate(accs))

            zero = jnp.zeros((L,), jnp.float32)
            accs = lax.fori_loop(0, steps, inner, (zero,) * ROWS_PER_CHUNK)
            for j in range(ROWS_PER_CHUNK):
                parts_v[slot, j, :] = accs[j]

            toff = c * ROWS_PER_CHUNK + jnp.where(maskr, lane, 0)
            tvec = plsc.load_gather(tgt_v, [toff], mask=maskr)
            vals = plsc.load_gather(buf, [lane, tvec], mask=maskr)
            parts_v[slot, ROWS_PER_CHUNK, :] = jnp.where(maskr, vals, 0.0)

        n_main = RING * ((cpw - PREF_D) // RING)
        def ring_body(p, carry):
            c = RING * p
            for j in range(RING):
                do_chunk(c + j, j, True, j)
            pltpu.sync_copy(parts_v, parts_hbm.at[pl.ds(cbase + c, RING)])
            return carry

        lax.fori_loop(0, n_main // RING, ring_body, 0)
        n_tail = cpw - n_main
        for c in range(n_main, cpw):
            do_chunk(c, c % RING, c + PREF_D < cpw, c - n_main)
        if n_tail:
            pltpu.sync_copy(parts_v.at[pl.ds(0, n_tail)],
                            parts_hbm.at[pl.ds(cbase + n_main, n_tail)])
        for c in range(cpw - RING, cpw):
            wait_scatter(c, c % RING)

    return body(table, idx2, tgt2)


def _tc_loss(parts, n_rows):
    def body(parts_ref, out_ref):
        p = parts_ref[...]
        lse = jnp.log(jnp.sum(p[:, :ROWS_PER_CHUNK, :], axis=2))
        total = ((jnp.sum(lse) - jnp.sum(p[:, ROWS_PER_CHUNK, :]))
                 / jnp.float32(n_rows))
        out_ref[...] = jnp.broadcast_to(total, (1, 1))

    return pl.pallas_call(
        body,
        out_shape=jax.ShapeDtypeStruct((1, 1), jnp.float32),
    )(parts)


def kernel(idx, target, table):
    b, s = idx.shape
    vocab = table.shape[1]
    n_rows = b * s

    idx2 = idx.reshape(-1).astype(jnp.int32).reshape(-1, ROWS_PER_CHUNK)
    tgt2 = target.reshape(-1).astype(jnp.int32)

    logits, parts = _sc_gather_loss(table, idx2, tgt2, n_rows, vocab)
    loss = _tc_loss(parts, n_rows)[0, 0]

    return (logits.reshape(b, s, vocab), loss)


# speedup vs baseline: 1.0067x; 1.0067x over previous
import functools

import jax
import jax.numpy as jnp
from jax import lax
from jax.experimental import pallas as pl
from jax.experimental.pallas import tpu as pltpu
from jax.experimental.pallas import tpu_sc as plsc

NC, NS, L = 2, 16, 16
NW = NC * NS

ROWS_PER_CHUNK = 4
RING = 3
PREF_D = 2


def _sc_gather_loss(table, idx2, tgt2, n_rows, vocab):
    n_chunks = n_rows // ROWS_PER_CHUNK
    cpw = n_chunks // NW
    steps = vocab // L

    mesh = plsc.VectorSubcoreMesh(
        core_axis_name="c", subcore_axis_name="s",
        num_cores=NC, num_subcores=NS)

    @functools.partial(
        pl.kernel,
        out_type=(
            jax.ShapeDtypeStruct((n_rows, vocab), jnp.float32),
            jax.ShapeDtypeStruct((n_chunks, ROWS_PER_CHUNK + 1, L), jnp.float32),
        ),
        mesh=mesh,
        compiler_params=pltpu.CompilerParams(needs_layout_passes=False),
        scratch_types=(
            [pltpu.VMEM((cpw, ROWS_PER_CHUNK), jnp.int32),
             pltpu.VMEM((cpw * ROWS_PER_CHUNK,), jnp.int32),
             pltpu.VMEM((ROWS_PER_CHUNK + 1, L), jnp.float32)]
            + [pltpu.VMEM((ROWS_PER_CHUNK, vocab), jnp.float32)] * RING
            + [pltpu.SemaphoreType.DMA] * (2 * RING)
        ),
    )
    def body(table_hbm, idx_hbm, tgt_hbm, logits_hbm, parts_hbm, *scratch):
        idx_v, tgt_v, parts_v = scratch[:3]
        bufs = scratch[3:3 + RING]
        gsems = scratch[3 + RING:3 + 2 * RING]
        ssems = scratch[3 + 2 * RING:3 + 3 * RING]

        w = lax.axis_index("s") * NC + lax.axis_index("c")
        cbase = w * cpw

        pltpu.sync_copy(idx_hbm.at[pl.ds(cbase, cpw)], idx_v)
        pltpu.sync_copy(
            tgt_hbm.at[pl.ds(cbase * ROWS_PER_CHUNK, cpw * ROWS_PER_CHUNK)],
            tgt_v)

        lane = lax.iota(jnp.int32, L)
        maskr = lane < ROWS_PER_CHUNK

        def start_gather(c, k):
            pltpu.async_copy(table_hbm.at[idx_v.at[c]], bufs[k], gsems[k])

        def wait_gather(c, k):
            pltpu.make_async_copy(
                table_hbm.at[idx_v.at[c]], bufs[k], gsems[k]).wait()

        def logits_dst(c):
            return logits_hbm.at[pl.ds((cbase + c) * ROWS_PER_CHUNK,
                                       ROWS_PER_CHUNK)]

        def wait_scatter(c, k):
            pltpu.make_async_copy(bufs[k], logits_dst(c), ssems[k]).wait()

        for m in range(PREF_D):
            start_gather(m, m)

        def do_chunk(c, k, prefetch):
            buf = bufs[k]
            wait_gather(c, k)

            pltpu.async_copy(buf, logits_dst(c), ssems[k])
            if prefetch:
                k2 = (k + PREF_D) % RING
                @pl.when(c >= RING - PREF_D)
                def _():
                    wait_scatter(c + PREF_D - RING, k2)
                start_gather(c + PREF_D, k2)

            def inner(i, accs):
                s = pl.ds(i * L, L)
                return tuple(a + jnp.exp(buf[j, s]) for j, a in enumerate(accs))

            zero = jnp.zeros((L,), jnp.float32)
            accs = lax.fori_loop(0, steps, inner, (zero,) * ROWS_PER_CHUNK)
            for j in range(ROWS_PER_CHUNK):
                parts_v[j, :] = accs[j]

            toff = c * ROWS_PER_CHUNK + jnp.where(maskr, lane, 0)
            tvec = plsc.load_gather(tgt_v, [toff], mask=maskr)
            vals = plsc.load_gather(buf, [lane, tvec], mask=maskr)
            parts_v[ROWS_PER_CHUNK, :] = jnp.where(maskr, vals, 0.0)
            pltpu.sync_copy(parts_v, parts_hbm.at[cbase + c])

        n_main = RING * ((cpw - PREF_D) // RING)
        def ring_body(p, carry):
            c = RING * p
            for j in range(RING):
                do_chunk(c + j, j, True)
            return carry

        lax.fori_loop(0, n_main // RING, ring_body, 0)
        for c in range(n_main, cpw):
            do_chunk(c, c % RING, c + PREF_D < cpw)
        for c in range(cpw - RING, cpw):
            wait_scatter(c, c % RING)

    return body(table, idx2, tgt2)


def _tc_loss(parts, n_rows):
    def body(parts_ref, out_ref):
        p = parts_ref[...]
        lse = jnp.log(jnp.sum(p[:, :ROWS_PER_CHUNK, :], axis=2))
        total = ((jnp.sum(lse) - jnp.sum(p[:, ROWS_PER_CHUNK, :]))
                 / jnp.float32(n_rows))
        out_ref[...] = jnp.broadcast_to(total, (1, 1))

    return pl.pallas_call(
        body,
        out_shape=jax.ShapeDtypeStruct((1, 1), jnp.float32),
    )(parts)


def kernel(idx, target, table):
    b, s = idx.shape
    vocab = table.shape[1]
    n_rows = b * s

    idx2 = idx.reshape(-1).astype(jnp.int32).reshape(-1, ROWS_PER_CHUNK)
    tgt2 = target.reshape(-1).astype(jnp.int32)

    logits, parts = _sc_gather_loss(table, idx2, tgt2, n_rows, vocab)
    loss = _tc_loss(parts, n_rows)[0, 0]

    return (logits.reshape(b, s, vocab), loss)


# speedup vs baseline: 1.0100x; 1.0033x over previous
import functools

import jax
import jax.numpy as jnp
from jax import lax
from jax.experimental import pallas as pl
from jax.experimental.pallas import tpu as pltpu
from jax.experimental.pallas import tpu_sc as plsc

NC, NS, L = 2, 16, 16
NW = NC * NS

ROWS_PER_CHUNK = 4
RING = 3
PREF_D = 2


def _sc_gather_loss(table, idx2, tgt2, n_rows, vocab):
    n_chunks = n_rows // ROWS_PER_CHUNK
    cpw = n_chunks // NW
    steps = vocab // L

    mesh = plsc.VectorSubcoreMesh(
        core_axis_name="c", subcore_axis_name="s",
        num_cores=NC, num_subcores=NS)

    @functools.partial(
        pl.kernel,
        out_type=(
            jax.ShapeDtypeStruct((n_rows, vocab), jnp.float32),
            jax.ShapeDtypeStruct((n_chunks, ROWS_PER_CHUNK + 1, L), jnp.float32),
        ),
        mesh=mesh,
        compiler_params=pltpu.CompilerParams(needs_layout_passes=False),
        scratch_types=(
            [pltpu.VMEM((cpw, ROWS_PER_CHUNK), jnp.int32),
             pltpu.VMEM((cpw * ROWS_PER_CHUNK,), jnp.int32),
             pltpu.VMEM((ROWS_PER_CHUNK + 1, L), jnp.float32)]
            + [pltpu.VMEM((ROWS_PER_CHUNK, vocab), jnp.float32)] * RING
            + [pltpu.SemaphoreType.DMA] * (2 * RING)
        ),
    )
    def body(table_hbm, idx_hbm, tgt_hbm, logits_hbm, parts_hbm, *scratch):
        idx_v, tgt_v, parts_v = scratch[:3]
        bufs = scratch[3:3 + RING]
        gsems = scratch[3 + RING:3 + 2 * RING]
        ssems = scratch[3 + 2 * RING:3 + 3 * RING]

        w = lax.axis_index("s") * NC + lax.axis_index("c")
        cbase = w * cpw

        pltpu.sync_copy(idx_hbm.at[pl.ds(cbase, cpw)], idx_v)
        pltpu.sync_copy(
            tgt_hbm.at[pl.ds(cbase * ROWS_PER_CHUNK, cpw * ROWS_PER_CHUNK)],
            tgt_v)

        lane = lax.iota(jnp.int32, L)
        maskr = lane < ROWS_PER_CHUNK

        def start_gather(c, k):
            pltpu.async_copy(table_hbm.at[idx_v.at[c]], bufs[k], gsems[k])

        def wait_gather(c, k):
            pltpu.make_async_copy(
                table_hbm.at[idx_v.at[c]], bufs[k], gsems[k]).wait()

        def logits_dst(c):
            return logits_hbm.at[pl.ds((cbase + c) * ROWS_PER_CHUNK,
                                       ROWS_PER_CHUNK)]

        def wait_scatter(c, k):
            pltpu.make_async_copy(bufs[k], logits_dst(c), ssems[k]).wait()

        for m in range(PREF_D):
            start_gather(m, m)

        def do_chunk(c, k, prefetch):
            buf = bufs[k]
            wait_gather(c, k)

            if prefetch:
                k2 = (k + PREF_D) % RING
                @pl.when(c >= RING - PREF_D)
                def _():
                    wait_scatter(c + PREF_D - RING, k2)
                start_gather(c + PREF_D, k2)
            pltpu.async_copy(buf, logits_dst(c), ssems[k])

            def inner(i, accs):
                s = pl.ds(i * L, L)
                return tuple(a + jnp.exp(buf[j, s]) for j, a in enumerate(accs))

            zero = jnp.zeros((L,), jnp.float32)
            accs = lax.fori_loop(0, steps, inner, (zero,) * ROWS_PER_CHUNK)
            for j in range(ROWS_PER_CHUNK):
                parts_v[j, :] = accs[j]

            toff = c * ROWS_PER_CHUNK + jnp.where(maskr, lane, 0)
            tvec = plsc.load_gather(tgt_v, [toff], mask=maskr)
            vals = plsc.load_gather(buf, [lane, tvec], mask=maskr)
            parts_v[ROWS_PER_CHUNK, :] = jnp.where(maskr, vals, 0.0)
            pltpu.sync_copy(parts_v, parts_hbm.at[cbase + c])

        n_main = RING * ((cpw - PREF_D) // RING)
        def ring_body(p, carry):
            c = RING * p
            for j in range(RING):
                do_chunk(c + j, j, True)
            return carry

        lax.fori_loop(0, n_main // RING, ring_body, 0)
        for c in range(n_main, cpw):
            do_chunk(c, c % RING, c + PREF_D < cpw)
        for c in range(cpw - RING, cpw):
            wait_scatter(c, c % RING)

    return body(table, idx2, tgt2)


def _tc_loss(parts, n_rows):
    def body(parts_ref, out_ref):
        p = parts_ref[...]
        lse = jnp.log(jnp.sum(p[:, :ROWS_PER_CHUNK, :], axis=2))
        total = ((jnp.sum(lse) - jnp.sum(p[:, ROWS_PER_CHUNK, :]))
                 / jnp.float32(n_rows))
        out_ref[...] = jnp.broadcast_to(total, (1, 1))

    return pl.pallas_call(
        body,
        out_shape=jax.ShapeDtypeStruct((1, 1), jnp.float32),
    )(parts)


def kernel(idx, target, table):
    b, s = idx.shape
    vocab = table.shape[1]
    n_rows = b * s

    idx2 = idx.reshape(-1).astype(jnp.int32).reshape(-1, ROWS_PER_CHUNK)
    tgt2 = target.reshape(-1).astype(jnp.int32)

    logits, parts = _sc_gather_loss(table, idx2, tgt2, n_rows, vocab)
    loss = _tc_loss(parts, n_rows)[0, 0]

    return (logits.reshape(b, s, vocab), loss)
